# Initial kernel scaffold; baseline (speedup 1.0000x reference)
import jax
import jax.numpy as jnp
from jax.experimental import pallas as pl


def kernel(head, relation, tail, negative, entity_embedding, relation_embedding):
    raise NotImplementedError("write your pallas kernel here")


# speedup vs baseline: 2.7000x; 2.7000x over previous
import functools

import jax
import jax.numpy as jnp
from jax import lax
from jax.experimental import pallas as pl
from jax.experimental.pallas import tpu as pltpu
from jax.experimental.pallas import tpu_sc as plsc

N_ENTITIES = 100000
DIM = 64
B = 2048
N_NEG = 128

NC = 2
NS = 16
L = 16
NW = NC * NS
BPW = B // NW
NCHUNK = N_NEG // L
NDBLK = DIM // L
NBUF = 4


def _sc_body(head_hbm, rel_hbm, tail_hbm, neg_hbm, ent_hbm, relemb_hbm,
             out_pos, out_neg,
             hidx_v, ridx_v, tidx_v, negidx_v,
             h_rows, r_rows, t_rows, hr_flat, negbuf, pos_v, nsc_v, qscr,
             sem_h, sem_r, sem_t, nsem0, nsem1, nsem2, nsem3):
    nsems = (nsem0, nsem1, nsem2, nsem3)
    wid = lax.axis_index("s") * NC + lax.axis_index("c")
    base = wid * BPW

    pltpu.sync_copy(neg_hbm.at[pl.ds(base, BPW)], negidx_v)
    pltpu.sync_copy(head_hbm.at[pl.ds(base, BPW)], hidx_v)
    pltpu.sync_copy(rel_hbm.at[pl.ds(base, BPW)], ridx_v)
    pltpu.sync_copy(tail_hbm.at[pl.ds(base, BPW)], tidx_v)

    for s in range(NBUF):
        pltpu.async_copy(ent_hbm.at[negidx_v.at[s]], negbuf.at[s], nsems[s])

    ch = pltpu.async_copy(ent_hbm.at[hidx_v], h_rows, sem_h)
    cr = pltpu.async_copy(relemb_hbm.at[ridx_v], r_rows, sem_r)
    ct = pltpu.async_copy(ent_hbm.at[tidx_v], t_rows, sem_t)
    ch.wait()
    cr.wait()
    ct.wait()

    iotav = lax.iota(jnp.int32, L)

    @pl.loop(0, BPW, step=L)
    def _hr_loop(g):
        for n in range(L):
            b = g + n
            q = jnp.zeros((L,), jnp.float32)
            for c in range(NDBLK):
                hv = h_rows[b, pl.ds(c * L, L)]
                rv = r_rows[b, pl.ds(c * L, L)]
                tv = t_rows[b, pl.ds(c * L, L)]
                hrv = hv * rv
                hr_flat[pl.ds(b * DIM + c * L, L)] = hrv
                q = q + hrv * tv
            qscr[pl.ds(n * L, L)] = q
        acc = jnp.zeros((L,), jnp.float32)
        for l in range(L):
            acc = acc + plsc.load_gather(qscr, [iotav * L + l])
        pos_v[pl.ds(g, L)] = acc
    rowvs = [iotav + c * L for c in range(NCHUNK)]

    @pl.loop(0, BPW, step=NBUF)
    def _main(b0):
        for s in range(NBUF):
            bb = b0 + s
            pltpu.make_async_copy(
                ent_hbm.at[negidx_v.at[bb]], negbuf.at[s], nsems[s]).wait()

            def dblk_body(t, accs):
                out = list(accs)
                hv16 = hr_flat[pl.ds(bb * DIM + t * L, L)]
                for j in range(L):
                    d = t * L + j
                    sp = jnp.full((L,), hv16[j])
                    colv = jnp.full((L,), d, jnp.int32)
                    for c in range(NCHUNK):
                        v = plsc.load_gather(negbuf.at[s], [rowvs[c], colv])
                        out[c] = out[c] + sp * v
                return tuple(out)

            accs = lax.fori_loop(
                0, NDBLK, dblk_body,
                tuple(jnp.zeros((L,), jnp.float32) for _ in range(NCHUNK)))
            for c in range(NCHUNK):
                nsc_v[bb, pl.ds(c * L, L)] = accs[c]

            @pl.when(bb + NBUF < BPW)
            def _():
                pltpu.async_copy(
                    ent_hbm.at[negidx_v.at[bb + NBUF]], negbuf.at[s],
                    nsems[s])

    pltpu.sync_copy(pos_v, out_pos.at[pl.ds(base, BPW)])
    pltpu.sync_copy(nsc_v, out_neg.at[pl.ds(base, BPW)])


@jax.jit
def _bess_scores(head, relation, tail, negative, entity_embedding,
                 relation_embedding):
    mesh = plsc.VectorSubcoreMesh(
        core_axis_name="c", subcore_axis_name="s",
        num_cores=NC, num_subcores=NS)
    kfn = pl.kernel(
        _sc_body,
        out_type=(
            jax.ShapeDtypeStruct((B,), jnp.float32),
            jax.ShapeDtypeStruct((B, N_NEG), jnp.float32),
        ),
        mesh=mesh,
        compiler_params=pltpu.CompilerParams(needs_layout_passes=False, use_tc_tiling_on_sc=False),
        scratch_types=[
            pltpu.VMEM((BPW,), jnp.int32),
            pltpu.VMEM((BPW,), jnp.int32),
            pltpu.VMEM((BPW,), jnp.int32),
            pltpu.VMEM((BPW, N_NEG), jnp.int32),
            pltpu.VMEM((BPW, DIM), jnp.float32),
            pltpu.VMEM((BPW, DIM), jnp.float32),
            pltpu.VMEM((BPW, DIM), jnp.float32),
            pltpu.VMEM((BPW * DIM,), jnp.float32),
            pltpu.VMEM((NBUF, N_NEG, DIM), jnp.float32),
            pltpu.VMEM((BPW,), jnp.float32),
            pltpu.VMEM((BPW, N_NEG), jnp.float32),
            pltpu.VMEM((L * L,), jnp.float32),
            pltpu.SemaphoreType.DMA,
            pltpu.SemaphoreType.DMA,
            pltpu.SemaphoreType.DMA,
            pltpu.SemaphoreType.DMA,
            pltpu.SemaphoreType.DMA,
            pltpu.SemaphoreType.DMA,
            pltpu.SemaphoreType.DMA,
        ],
    )
    return kfn(head, relation, tail, negative, entity_embedding,
               relation_embedding)


def kernel(head, relation, tail, negative, entity_embedding,
           relation_embedding):
    pos, negs = _bess_scores(
        head.reshape(B), relation.reshape(B), tail.reshape(B),
        negative.reshape(B, N_NEG), entity_embedding, relation_embedding)
    return jnp.concatenate([pos[:, None], negs], axis=1)


# speedup vs baseline: 6.2912x; 2.3300x over previous
import functools

import jax
import jax.numpy as jnp
from jax import lax
from jax.experimental import pallas as pl
from jax.experimental.pallas import tpu as pltpu
from jax.experimental.pallas import tpu_sc as plsc

N_ENTITIES = 100000
DIM = 64
B = 2048
N_NEG = 128

NC = 2
NS = 16
L = 16
NW = NC * NS
BPW = B // NW
NCHUNK = N_NEG // L
NDBLK = DIM // L
NBUF = 4
QP = 17


def _sc_body(head_hbm, rel_hbm, tail_hbm, neg_hbm, ent_hbm, relemb_hbm,
             out_pos, out_neg,
             hidx_v, ridx_v, tidx_v, negidx_v,
             h_rows, r_rows, t_rows, hr_flat, negbuf, pos_v, nsc_v, qscr,
             sem_h, sem_r, sem_t, nsem0, nsem1, nsem2, nsem3):
    nsems = (nsem0, nsem1, nsem2, nsem3)
    wid = lax.axis_index("s") * NC + lax.axis_index("c")
    base = wid * BPW

    pltpu.sync_copy(neg_hbm.at[pl.ds(base, BPW)], negidx_v)
    pltpu.sync_copy(head_hbm.at[pl.ds(base, BPW)], hidx_v)
    pltpu.sync_copy(rel_hbm.at[pl.ds(base, BPW)], ridx_v)
    pltpu.sync_copy(tail_hbm.at[pl.ds(base, BPW)], tidx_v)

    for s in range(NBUF):
        pltpu.async_copy(ent_hbm.at[negidx_v.at[s]], negbuf.at[s], nsems[s])

    ch = pltpu.async_copy(ent_hbm.at[hidx_v], h_rows, sem_h)
    cr = pltpu.async_copy(relemb_hbm.at[ridx_v], r_rows, sem_r)
    ct = pltpu.async_copy(ent_hbm.at[tidx_v], t_rows, sem_t)
    ch.wait()
    cr.wait()
    ct.wait()

    iotav = lax.iota(jnp.int32, L)
    iq = iotav * QP

    @pl.loop(0, BPW, step=L)
    def _hr_loop(g):
        for n in range(L):
            b = g + n
            q = jnp.zeros((L,), jnp.float32)
            for c in range(NDBLK):
                hv = h_rows[b, pl.ds(c * L, L)]
                rv = r_rows[b, pl.ds(c * L, L)]
                tv = t_rows[b, pl.ds(c * L, L)]
                hrv = hv * rv
                hr_flat[pl.ds(b * DIM + c * L, L)] = hrv
                q = q + hrv * tv
            plsc.store_scatter(qscr, [iq + n], q)
        acc = jnp.zeros((L,), jnp.float32)
        for l in range(L):
            acc = acc + qscr[pl.ds(l * QP, L)]
        pos_v[pl.ds(g, L)] = acc

    @pl.loop(0, BPW, step=NBUF)
    def _main(b0):
        for s in range(NBUF):
            bb = b0 + s
            pltpu.make_async_copy(
                ent_hbm.at[negidx_v.at[bb]], negbuf.at[s], nsems[s]).wait()

            hrvs = [hr_flat[pl.ds(bb * DIM + t * L, L)]
                    for t in range(NDBLK)]

            @pl.loop(0, NCHUNK)
            def _chunk(c):
                for n in range(L):
                    row = c * L + n
                    p = jnp.zeros((L,), jnp.float32)
                    for t in range(NDBLK):
                        v = negbuf[s, row, pl.ds(t * L, L)]
                        p = p + hrvs[t] * v
                    plsc.store_scatter(qscr, [iq + n], p)
                acc = jnp.zeros((L,), jnp.float32)
                for l in range(L):
                    acc = acc + qscr[pl.ds(l * QP, L)]
                nsc_v[bb, pl.ds(c * L, L)] = acc

            @pl.when(bb + NBUF < BPW)
            def _():
                pltpu.async_copy(
                    ent_hbm.at[negidx_v.at[bb + NBUF]], negbuf.at[s],
                    nsems[s])

    pltpu.sync_copy(pos_v, out_pos.at[pl.ds(base, BPW)])
    pltpu.sync_copy(nsc_v, out_neg.at[pl.ds(base, BPW)])


@jax.jit
def _bess_scores(head, relation, tail, negative, entity_embedding,
                 relation_embedding):
    mesh = plsc.VectorSubcoreMesh(
        core_axis_name="c", subcore_axis_name="s",
        num_cores=NC, num_subcores=NS)
    kfn = pl.kernel(
        _sc_body,
        out_type=(
            jax.ShapeDtypeStruct((B,), jnp.float32),
            jax.ShapeDtypeStruct((B, N_NEG), jnp.float32),
        ),
        mesh=mesh,
        compiler_params=pltpu.CompilerParams(needs_layout_passes=False, use_tc_tiling_on_sc=False),
        scratch_types=[
            pltpu.VMEM((BPW,), jnp.int32),
            pltpu.VMEM((BPW,), jnp.int32),
            pltpu.VMEM((BPW,), jnp.int32),
            pltpu.VMEM((BPW, N_NEG), jnp.int32),
            pltpu.VMEM((BPW, DIM), jnp.float32),
            pltpu.VMEM((BPW, DIM), jnp.float32),
            pltpu.VMEM((BPW, DIM), jnp.float32),
            pltpu.VMEM((BPW * DIM,), jnp.float32),
            pltpu.VMEM((NBUF, N_NEG, DIM), jnp.float32),
            pltpu.VMEM((BPW,), jnp.float32),
            pltpu.VMEM((BPW, N_NEG), jnp.float32),
            pltpu.VMEM((L * QP,), jnp.float32),
            pltpu.SemaphoreType.DMA,
            pltpu.SemaphoreType.DMA,
            pltpu.SemaphoreType.DMA,
            pltpu.SemaphoreType.DMA,
            pltpu.SemaphoreType.DMA,
            pltpu.SemaphoreType.DMA,
            pltpu.SemaphoreType.DMA,
        ],
    )
    return kfn(head, relation, tail, negative, entity_embedding,
               relation_embedding)


def kernel(head, relation, tail, negative, entity_embedding,
           relation_embedding):
    pos, negs = _bess_scores(
        head.reshape(B), relation.reshape(B), tail.reshape(B),
        negative.reshape(B, N_NEG), entity_embedding, relation_embedding)
    return jnp.concatenate([pos[:, None], negs], axis=1)


# speedup vs baseline: 11.0989x; 1.7642x over previous
import functools

import jax
import jax.numpy as jnp
from jax import lax
from jax.experimental import pallas as pl
from jax.experimental.pallas import tpu as pltpu
from jax.experimental.pallas import tpu_sc as plsc

N_ENTITIES = 100000
DIM = 64
B = 2048
N_NEG = 128

NC = 2
NS = 16
L = 16
NW = NC * NS
BPW = B // NW
NCHUNK = N_NEG // L
NDBLK = DIM // L
NBUF = 4
QP = 17
DP = 128


def _sc_body(head_hbm, rel_hbm, tail_hbm, neg_hbm, ent_hbm, relemb_hbm,
             out_pos, out_neg,
             hidx_v, ridx_v, tidx_v, negidx_v,
             h_rows, r_rows, t_rows, hr_flat, negbuf, pos_v, nsc_v, qscr, qscr2,
             sem_h, sem_r, sem_t, nsem0, nsem1, nsem2, nsem3):
    nsems = (nsem0, nsem1, nsem2, nsem3)
    wid = lax.axis_index("s") * NC + lax.axis_index("c")
    base = wid * BPW

    pltpu.sync_copy(neg_hbm.at[pl.ds(base, BPW)], negidx_v)
    pltpu.sync_copy(head_hbm.at[pl.ds(base, BPW)], hidx_v)
    pltpu.sync_copy(rel_hbm.at[pl.ds(base, BPW)], ridx_v)
    pltpu.sync_copy(tail_hbm.at[pl.ds(base, BPW)], tidx_v)

    @pl.loop(0, BPW)
    def _dblneg(i):
        for c in range(N_NEG // L):
            negidx_v[i, pl.ds(c * L, L)] = negidx_v[i, pl.ds(c * L, L)] * 2

    @pl.loop(0, BPW // L)
    def _dblht(i):
        hidx_v[pl.ds(i * L, L)] = hidx_v[pl.ds(i * L, L)] * 2
        tidx_v[pl.ds(i * L, L)] = tidx_v[pl.ds(i * L, L)] * 2

    for s in range(NBUF):
        pltpu.async_copy(ent_hbm.at[negidx_v.at[s]], negbuf.at[s], nsems[s])

    ch = pltpu.async_copy(ent_hbm.at[hidx_v], h_rows, sem_h)
    cr = pltpu.async_copy(relemb_hbm.at[ridx_v], r_rows, sem_r)
    ct = pltpu.async_copy(ent_hbm.at[tidx_v], t_rows, sem_t)
    ch.wait()
    cr.wait()
    ct.wait()

    iotav = lax.iota(jnp.int32, L)
    iq = iotav * QP

    @pl.loop(0, BPW, step=L)
    def _hr_loop(g):
        for n in range(L):
            b = g + n
            q = jnp.zeros((L,), jnp.float32)
            for c in range(NDBLK):
                hv = h_rows[b, pl.ds(c * L, L)]
                rv = r_rows[b, pl.ds(c * L, L)]
                tv = t_rows[b, pl.ds(c * L, L)]
                hrv = hv * rv
                hr_flat[pl.ds(b * DIM + c * L, L)] = hrv
                q = q + hrv * tv
            plsc.store_scatter(qscr, [iq + n], q)
        acc = jnp.zeros((L,), jnp.float32)
        for l in range(L):
            acc = acc + qscr[pl.ds(l * QP, L)]
        pos_v[pl.ds(g, L)] = acc

    @pl.loop(0, BPW, step=NBUF)
    def _main(b0):
        for s in range(NBUF):
            bb = b0 + s
            pltpu.make_async_copy(
                ent_hbm.at[pl.ds(0, N_NEG)], negbuf.at[s], nsems[s]).wait()

            hrvs = [hr_flat[pl.ds(bb * DIM + t * L, L)]
                    for t in range(NDBLK)]

            @pl.loop(0, NCHUNK, step=2)
            def _chunk(c):
                qrefs = (qscr, qscr2)
                row0 = c * L
                vs = [negbuf[s, row0, pl.ds(t * L, L)]
                      for t in range(NDBLK)]
                vs1 = [negbuf[s, row0 + 1, pl.ds(t * L, L)]
                       for t in range(NDBLK)]
                for n in range(2 * L):
                    if n >= 2 * L - 2:
                        nrow = jnp.minimum(row0 + n + 2, N_NEG - 1)
                    else:
                        nrow = row0 + n + 2
                    nxt = [negbuf[s, nrow, pl.ds(t * L, L)]
                           for t in range(NDBLK)]
                    ms = [hrvs[t] * vs[t] for t in range(NDBLK)]
                    plsc.store_scatter(
                        qrefs[n // L], [iq + (n % L)],
                        (ms[0] + ms[1]) + (ms[2] + ms[3]))
                    vs = vs1
                    vs1 = nxt
                for half in range(2):
                    rows = [qrefs[half][pl.ds(l * QP, L)] for l in range(L)]
                    while len(rows) > 1:
                        rows = [rows[i] + rows[i + 1]
                                for i in range(0, len(rows), 2)]
                    nsc_v[bb, pl.ds((c + half) * L, L)] = rows[0]

            @pl.when(bb + NBUF < BPW)
            def _():
                pltpu.async_copy(
                    ent_hbm.at[negidx_v.at[bb + NBUF]], negbuf.at[s],
                    nsems[s])

    pltpu.sync_copy(pos_v, out_pos.at[pl.ds(base, BPW)])
    pltpu.sync_copy(nsc_v, out_neg.at[pl.ds(base, BPW)])


NBLK = (N_ENTITIES + 127) // 128


def _fmt_body(entT_hbm, out_hbm, ibuf0, ibuf1, obuf0, obuf1,
              isem0, isem1, osem0, osem1):
    wid = lax.axis_index("s") * NC + lax.axis_index("c")
    ibufs = (ibuf0, ibuf1)
    isems = (isem0, isem1)
    obufs = (obuf0, obuf1)
    osems = (osem0, osem1)
    iotav = lax.iota(jnp.int32, L)

    def start_in(jj, p):
        blk = wid + NW * jj

        @pl.when(blk < NBLK)
        def _():
            eb = pl.multiple_of(blk * 128, 128)
            pltpu.async_copy(entT_hbm.at[:, pl.ds(eb, 128)], ibufs[p],
                             isems[p])

    start_in(0, 0)
    start_in(1, 1)

    @pl.loop(0, 26, step=2)
    def _blocks(j):
        for p in range(2):
            jj = j + p
            blk = wid + NW * jj

            @pl.when(blk < NBLK)
            def _():
                eb = pl.multiple_of(blk * 128, 128)
                pltpu.make_async_copy(
                    entT_hbm.at[:, pl.ds(0, 128)], ibufs[p],
                    isems[p]).wait()

                @pl.when(jj >= 2)
                def _():
                    pltpu.make_async_copy(
                        obufs[p], out_hbm.at[pl.ds(0, 128)],
                        osems[p]).wait()

                @pl.loop(0, L, step=2)
                def _k(k0):
                    prev = None
                    for dk in range(2):
                        k = k0 + dk
                        dperm = (iotav + k) & (L - 1)
                        for g in range(8):
                            rowv = dperm + g * L
                            for t in range(NDBLK):
                                colv = iotav + t * L
                                v = plsc.load_gather(
                                    ibufs[p], [colv, rowv])
                                if prev is not None:
                                    plsc.store_scatter(
                                        obufs[p], [prev[1], prev[2]],
                                        prev[0])
                                prev = (v, rowv, colv)
                    plsc.store_scatter(obufs[p], [prev[1], prev[2]],
                                       prev[0])

                pltpu.async_copy(obufs[p], out_hbm.at[pl.ds(eb, 128)],
                                 osems[p])
                start_in(jj + 2, p)

    for p in range(2):
        pltpu.make_async_copy(
            obufs[p], out_hbm.at[pl.ds(0, 128)], osems[p]).wait()


@jax.jit
def _format_table(entT):
    mesh = plsc.VectorSubcoreMesh(
        core_axis_name="c", subcore_axis_name="s",
        num_cores=NC, num_subcores=NS)
    kfn = pl.kernel(
        _fmt_body,
        out_type=jax.ShapeDtypeStruct((NBLK * 128, DP), jnp.float32),
        mesh=mesh,
        compiler_params=pltpu.CompilerParams(
            needs_layout_passes=False, use_tc_tiling_on_sc=True),
        scratch_types=[
            pltpu.VMEM((DIM, 128), jnp.float32),
            pltpu.VMEM((DIM, 128), jnp.float32),
            pltpu.VMEM((128, 128), jnp.float32),
            pltpu.VMEM((128, 128), jnp.float32),
            pltpu.SemaphoreType.DMA,
            pltpu.SemaphoreType.DMA,
            pltpu.SemaphoreType.DMA,
            pltpu.SemaphoreType.DMA,
        ],
    )
    return kfn(entT)


@jax.jit
def _bess_scores(head, relation, tail, negative, entity_embedding,
                 relation_embedding):
    mesh = plsc.VectorSubcoreMesh(
        core_axis_name="c", subcore_axis_name="s",
        num_cores=NC, num_subcores=NS)
    kfn = pl.kernel(
        _sc_body,
        out_type=(
            jax.ShapeDtypeStruct((B,), jnp.float32),
            jax.ShapeDtypeStruct((B, N_NEG), jnp.float32),
        ),
        mesh=mesh,
        compiler_params=pltpu.CompilerParams(needs_layout_passes=False, use_tc_tiling_on_sc=False),
        scratch_types=[
            pltpu.VMEM((BPW,), jnp.int32),
            pltpu.VMEM((BPW,), jnp.int32),
            pltpu.VMEM((BPW,), jnp.int32),
            pltpu.VMEM((BPW, N_NEG), jnp.int32),
            pltpu.VMEM((BPW, DIM), jnp.float32),
            pltpu.VMEM((BPW, DIM), jnp.float32),
            pltpu.VMEM((BPW, DIM), jnp.float32),
            pltpu.VMEM((BPW * DIM,), jnp.float32),
            pltpu.VMEM((NBUF, N_NEG, DIM), jnp.float32),
            pltpu.VMEM((BPW,), jnp.float32),
            pltpu.VMEM((BPW, N_NEG), jnp.float32),
            pltpu.VMEM((L * QP,), jnp.float32),
            pltpu.VMEM((L * QP,), jnp.float32),
            pltpu.SemaphoreType.DMA,
            pltpu.SemaphoreType.DMA,
            pltpu.SemaphoreType.DMA,
            pltpu.SemaphoreType.DMA,
            pltpu.SemaphoreType.DMA,
            pltpu.SemaphoreType.DMA,
            pltpu.SemaphoreType.DMA,
        ],
    )
    return kfn(head, relation, tail, negative, entity_embedding,
               relation_embedding)


def kernel(head, relation, tail, negative, entity_embedding,
           relation_embedding):
    ent_pad = _format_table(entity_embedding.T).reshape(-1, DIM)
    pos, negs = _bess_scores(
        head.reshape(B), relation.reshape(B), tail.reshape(B),
        negative.reshape(B, N_NEG), ent_pad, relation_embedding)
    return jnp.concatenate([pos[:, None], negs], axis=1)
